# Initial kernel scaffold; baseline (speedup 1.0000x reference)
import jax
import jax.numpy as jnp
from jax.experimental import pallas as pl


def kernel(x, edge_index, W, a_src, a_dst, W_skip, bias):
    raise NotImplementedError("write your pallas kernel here")


# speedup vs baseline: 6.4379x; 6.4379x over previous
import functools

import jax
import jax.numpy as jnp
from jax import lax
from jax.experimental import pallas as pl
from jax.experimental.pallas import tpu as pltpu
from jax.experimental.pallas import tpu_sc as plsc

N = 10000
NP = 10112
E = 160000
EP = 163840
NH = 8
NW = 32
EPW = EP // NW
BE_A = 1024
NB_A = EPW // BE_A
BE_B = 256
EPW_B = EP // 16
NB_B2 = EPW_B // BE_B
NB_B = EPW // BE_B
NCHUNK = 4
RB = 400
RPS = NP // 16


def _dense_body(x_ref, w_ref, wsk_ref, asrc_ref, adst_ref, bias_ref,
                *out_refs):
    (p0_ref, p1_ref, p2_ref, p3_ref, ss_ref, sd_ref, skip_ref) = out_refs
    xb = x_ref[...]
    pr = jnp.dot(xb, w_ref[...], preferred_element_type=jnp.float32)
    for c, pref in enumerate((p0_ref, p1_ref, p2_ref, p3_ref)):
        pref[...] = pr[:, c * 128:(c + 1) * 128]
    ss_ref[...] = jnp.dot(pr, asrc_ref[...], preferred_element_type=jnp.float32)
    sd_ref[...] = jnp.dot(pr, adst_ref[...], preferred_element_type=jnp.float32)
    skip_ref[...] = (jnp.dot(xb, wsk_ref[...], preferred_element_type=jnp.float32)
                     + bias_ref[...])


def _dense(x, W, W_skip, A_src, A_dst, bias2d):
    f32 = jnp.float32
    return pl.pallas_call(
        _dense_body,
        grid=(N // RB,),
        in_specs=[
            pl.BlockSpec((RB, 256), lambda i: (i, 0)),
            pl.BlockSpec((256, 512), lambda i: (0, 0)),
            pl.BlockSpec((256, 512), lambda i: (0, 0)),
            pl.BlockSpec((512, 8), lambda i: (0, 0)),
            pl.BlockSpec((512, 8), lambda i: (0, 0)),
            pl.BlockSpec((1, 512), lambda i: (0, 0)),
        ],
        out_specs=[pl.BlockSpec((RB, 128), lambda i: (i, 0))] * 4 + [
            pl.BlockSpec((RB, 8), lambda i: (i, 0)),
            pl.BlockSpec((RB, 8), lambda i: (i, 0)),
            pl.BlockSpec((RB, 512), lambda i: (i, 0)),
        ],
        out_shape=[jax.ShapeDtypeStruct((N, 128), f32)] * 4 + [
            jax.ShapeDtypeStruct((N, 8), f32),
            jax.ShapeDtypeStruct((N, 8), f32),
            jax.ShapeDtypeStruct((N, 512), f32),
        ],
    )(x, W, W_skip, A_src, A_dst, bias2d)


def _scores_body(sidxf_hbm, didxf_hbm, ssf_hbm, sdf_hbm, z8_hbm,
                 p_hbm, den_hbm,
                 sidx_v, didx_v, ra_v, rb_v, p_v, z_v, acc, sem):
    cc = lax.axis_index("c")
    sid = lax.axis_index("s")
    wid = cc * 16 + sid
    frow0 = pl.multiple_of(sid * (RPS * 8), 8)
    pltpu.sync_copy(z8_hbm.at[pl.ds(frow0, RPS * 8)], z_v)
    pltpu.sync_copy(z_v, acc.at[pl.ds(frow0, RPS * 8)])
    plsc.subcore_barrier()
    for b in range(NB_A):
        base = pl.multiple_of(wid * EPW + b * BE_A, 8)
        fbase = pl.multiple_of(base * 8, 8)
        pltpu.sync_copy(sidxf_hbm.at[pl.ds(fbase, BE_A * 8)], sidx_v)
        pltpu.sync_copy(didxf_hbm.at[pl.ds(fbase, BE_A * 8)], didx_v)
        pltpu.async_copy(ssf_hbm.at[sidx_v], ra_v, sem).wait()
        pltpu.async_copy(sdf_hbm.at[didx_v], rb_v, sem).wait()

        def body(j, _):
            v = ra_v[pl.ds(j * 16, 16)] + rb_v[pl.ds(j * 16, 16)]
            v = jnp.maximum(v, 0.2 * v)
            p_v[pl.ds(j * 16, 16)] = jnp.exp(v)
            return 0

        lax.fori_loop(0, BE_A * 8 // 16, body, 0)
        pltpu.sync_copy(p_v, p_hbm.at[pl.ds(fbase, BE_A * 8)])
        pltpu.sync_copy(p_v, acc.at[didx_v], add=True)
    plsc.subcore_barrier()
    dof = pl.multiple_of(cc * (NP * 8) + frow0, 8)
    pltpu.sync_copy(acc.at[pl.ds(frow0, RPS * 8)], z_v)
    pltpu.sync_copy(z_v, den_hbm.at[pl.ds(dof, RPS * 8)])


def _scores(sidx_f, didx_f, ssf, sdf, z8):
    f32 = jnp.float32
    mesh = plsc.VectorSubcoreMesh(core_axis_name="c", subcore_axis_name="s")
    k = functools.partial(
        pl.kernel,
        mesh=mesh,
        out_type=[jax.ShapeDtypeStruct((EP * 8,), f32),
                  jax.ShapeDtypeStruct((2 * NP * 8,), f32)],
        scratch_types=[
            pltpu.VMEM((BE_A * 8,), jnp.int32),
            pltpu.VMEM((BE_A * 8,), jnp.int32),
            pltpu.VMEM((BE_A * 8,), f32),
            pltpu.VMEM((BE_A * 8,), f32),
            pltpu.VMEM((BE_A * 8,), f32),
            pltpu.VMEM((RPS * 8,), f32),
            pltpu.VMEM_SHARED((NP * 8,), f32),
            pltpu.SemaphoreType.DMA,
        ],
    )(_scores_body)
    return k(sidx_f, didx_f, ssf, sdf, z8)


def _attn_body(didxf_hbm, p_hbm, d0_hbm, d1_hbm,
               attn_hbm,
               didx_v, r0_v, r1_v, p_v, at_v, sem):
    cc = lax.axis_index("c")
    sid = lax.axis_index("s")
    wid = cc * 16 + sid
    for b in range(NB_A):
        base = pl.multiple_of(wid * EPW + b * BE_A, 8)
        fbase = pl.multiple_of(base * 8, 8)
        pltpu.sync_copy(didxf_hbm.at[pl.ds(fbase, BE_A * 8)], didx_v)
        pltpu.async_copy(d0_hbm.at[didx_v], r0_v, sem).wait()
        pltpu.async_copy(d1_hbm.at[didx_v], r1_v, sem).wait()
        pltpu.sync_copy(p_hbm.at[pl.ds(fbase, BE_A * 8)], p_v)

        def body(j, _):
            sl = pl.ds(j * 16, 16)
            at_v[sl] = p_v[sl] / (r0_v[sl] + r1_v[sl] + 1e-16)
            return 0

        lax.fori_loop(0, BE_A * 8 // 16, body, 0)
        pltpu.sync_copy(at_v, attn_hbm.at[pl.ds(fbase, BE_A * 8)])


def _attn(didx_f, p_hbm, den0, den1):
    f32 = jnp.float32
    mesh = plsc.VectorSubcoreMesh(core_axis_name="c", subcore_axis_name="s")
    k = functools.partial(
        pl.kernel,
        mesh=mesh,
        out_type=jax.ShapeDtypeStruct((EP * 8,), f32),
        scratch_types=[
            pltpu.VMEM((BE_A * 8,), jnp.int32),
            pltpu.VMEM((BE_A * 8,), f32),
            pltpu.VMEM((BE_A * 8,), f32),
            pltpu.VMEM((BE_A * 8,), f32),
            pltpu.VMEM((BE_A * 8,), f32),
            pltpu.SemaphoreType.DMA,
        ],
    )(_attn_body)
    return k(didx_f, p_hbm, den0, den1)


def _agg_body(src_hbm, dst_hbm, attn_hbm, *rest):
    projs = rest[:NCHUNK]
    z128_hbm = rest[NCHUNK]
    outs = rest[NCHUNK + 1:2 * NCHUNK + 1]
    src_v, dst_v, at_v, proj_v, acc, sem = rest[2 * NCHUNK + 1:]
    sid = lax.axis_index("s")
    wid = sid
    row0 = pl.multiple_of(sid * RPS, 8)
    for c in range(NCHUNK):
        pltpu.sync_copy(z128_hbm.at[pl.ds(row0, RPS)], acc.at[pl.ds(row0, RPS)])
        plsc.subcore_barrier()

        def batch(b, _, c=c):
            base = pl.multiple_of(wid * EPW_B + b * BE_B, 8)
            fbase = pl.multiple_of(base * 8, 8)
            pltpu.sync_copy(src_hbm.at[pl.ds(base, BE_B)], src_v)
            pltpu.sync_copy(dst_hbm.at[pl.ds(base, BE_B)], dst_v)
            pltpu.sync_copy(attn_hbm.at[pl.ds(fbase, BE_B * 8)],
                            at_v.at[pl.ds(0, BE_B * 8)])
            pltpu.async_copy(projs[c].at[src_v], proj_v, sem).wait()

            def body(j, _):
                for l in range(16):
                    e = j * 16 + l
                    for hh in range(2):
                        w = at_v[pl.ds(e * 8 + 2 * c + hh, 16)]
                        bvec = jnp.broadcast_to(w[0], (16,))
                        for kq in range(4):
                            off = hh * 64 + kq * 16
                            proj_v[e, pl.ds(off, 16)] = (
                                proj_v[e, pl.ds(off, 16)] * bvec)
                return 0

            lax.fori_loop(0, BE_B // 16, body, 0)
            pltpu.sync_copy(proj_v, acc.at[dst_v], add=True)
            return 0

        lax.fori_loop(0, NB_B2, batch, 0)
        plsc.subcore_barrier()
        pltpu.sync_copy(acc.at[pl.ds(row0, RPS)], outs[c].at[pl.ds(row0, RPS)])
        plsc.subcore_barrier()


def _aggregate(src_p, dst_p, attn_hbm, projs, z128):
    f32 = jnp.float32
    mesh = plsc.VectorSubcoreMesh(core_axis_name="c", subcore_axis_name="s",
                                  num_cores=1)
    k = functools.partial(
        pl.kernel,
        mesh=mesh,
        out_type=[jax.ShapeDtypeStruct((NP, 128), f32)] * NCHUNK,
        scratch_types=[
            pltpu.VMEM((BE_B,), jnp.int32),
            pltpu.VMEM((BE_B,), jnp.int32),
            pltpu.VMEM((BE_B * 8 + 16,), f32),
            pltpu.VMEM((BE_B, 128), f32),
            pltpu.VMEM_SHARED((NP, 128), f32),
            pltpu.SemaphoreType.DMA,
        ],
    )(_agg_body)
    return k(src_p, dst_p, attn_hbm, *projs, z128)


def _final_body(*refs):
    parts = refs[:NCHUNK]
    skip_ref, o_ref = refs[NCHUNK:]
    for c, pr in enumerate(parts):
        s = pr[...] + skip_ref[:, c * 128:(c + 1) * 128]
        o_ref[:, c * 128:(c + 1) * 128] = jnp.where(
            s > 0, s, jnp.exp(jnp.minimum(s, 0.0)) - 1.0)


def _final(parts, skip):
    return pl.pallas_call(
        _final_body,
        grid=(N // RB,),
        in_specs=[pl.BlockSpec((RB, 128), lambda i: (i, 0))] * NCHUNK
        + [pl.BlockSpec((RB, 512), lambda i: (i, 0))],
        out_specs=pl.BlockSpec((RB, 512), lambda i: (i, 0)),
        out_shape=jax.ShapeDtypeStruct((N, 512), jnp.float32),
    )(*parts, skip)


def kernel(x, edge_index, W, a_src, a_dst, W_skip, bias):
    f32 = jnp.float32
    eye = jnp.eye(NH, dtype=f32)
    A_src = (a_src[:, :, None] * eye[:, None, :]).reshape(NH * 64, NH)
    A_dst = (a_dst[:, :, None] * eye[:, None, :]).reshape(NH * 64, NH)
    bias2d = bias.reshape(1, 512)

    dense_outs = _dense(x, W, W_skip, A_src, A_dst, bias2d)
    projs = tuple(dense_outs[:4])
    ssrc, sdst, skip = dense_outs[4:]

    pad = EP - E
    src_p = jnp.concatenate([edge_index[0], jnp.zeros((pad,), jnp.int32)])
    dst_p = jnp.concatenate([edge_index[1], jnp.full((pad,), N + 16, jnp.int32)])
    ar8 = jnp.arange(8, dtype=jnp.int32)
    sidx_f = (src_p[:, None] * 8 + ar8).reshape(-1)
    didx_f = (dst_p[:, None] * 8 + ar8).reshape(-1)
    z8 = jnp.zeros((NP * 8,), f32)
    z128 = jnp.zeros((NP, 128), f32)

    padn = (NP - N) * 8
    ssf = jnp.pad(ssrc.reshape(-1), (0, padn))
    sdf = jnp.pad(sdst.reshape(-1), (0, padn))
    p_hbm, den = _scores(sidx_f, didx_f, ssf, sdf, z8)
    attn_hbm = _attn(didx_f, p_hbm, den[:NP * 8], den[NP * 8:])
    parts = _aggregate(src_p, dst_p, attn_hbm, projs, z128)
    return _final(parts, skip)


# speedup vs baseline: 7.8971x; 1.2266x over previous
import functools

import jax
import jax.numpy as jnp
from jax import lax
from jax.experimental import pallas as pl
from jax.experimental.pallas import tpu as pltpu
from jax.experimental.pallas import tpu_sc as plsc

N = 10000
NP = 10112
E = 160000
EP = 163840
NH = 8
NW = 32
EPW = EP // NW
BE_A = 1024
NB_A = EPW // BE_A
BE_B = 256
NB_B2 = EPW // BE_B
NCHUNK = 4
RB = 400
RPS = NP // 16


def _dense_body(x_ref, w_ref, wsk_ref, asrc_ref, adst_ref, bias_ref,
                *out_refs):
    (p0_ref, p1_ref, p2_ref, p3_ref, ss_ref, sd_ref, skip_ref) = out_refs
    xb = x_ref[...]
    pr = jnp.dot(xb, w_ref[...], preferred_element_type=jnp.float32)
    for c, pref in enumerate((p0_ref, p1_ref, p2_ref, p3_ref)):
        pref[...] = pr[:, c * 128:(c + 1) * 128]
    ss_ref[...] = jnp.dot(pr, asrc_ref[...], preferred_element_type=jnp.float32)
    sd_ref[...] = jnp.dot(pr, adst_ref[...], preferred_element_type=jnp.float32)
    skip_ref[...] = (jnp.dot(xb, wsk_ref[...], preferred_element_type=jnp.float32)
                     + bias_ref[...])


def _dense(x, W, W_skip, A_src, A_dst, bias2d):
    f32 = jnp.float32
    return pl.pallas_call(
        _dense_body,
        grid=(N // RB,),
        in_specs=[
            pl.BlockSpec((RB, 256), lambda i: (i, 0)),
            pl.BlockSpec((256, 512), lambda i: (0, 0)),
            pl.BlockSpec((256, 512), lambda i: (0, 0)),
            pl.BlockSpec((512, 8), lambda i: (0, 0)),
            pl.BlockSpec((512, 8), lambda i: (0, 0)),
            pl.BlockSpec((1, 512), lambda i: (0, 0)),
        ],
        out_specs=[pl.BlockSpec((RB, 128), lambda i: (i, 0))] * 4 + [
            pl.BlockSpec((RB, 8), lambda i: (i, 0)),
            pl.BlockSpec((RB, 8), lambda i: (i, 0)),
            pl.BlockSpec((RB, 512), lambda i: (i, 0)),
        ],
        out_shape=[jax.ShapeDtypeStruct((N, 128), f32)] * 4 + [
            jax.ShapeDtypeStruct((N, 8), f32),
            jax.ShapeDtypeStruct((N, 8), f32),
            jax.ShapeDtypeStruct((N, 512), f32),
        ],
    )(x, W, W_skip, A_src, A_dst, bias2d)


def _scores_body(sidxf_hbm, didxf_hbm, ssf_hbm, sdf_hbm, z8_hbm,
                 p_hbm, den_hbm,
                 sidx_v, didx_v, ra_v, rb_v, p_v, z_v, acc, sem):
    cc = lax.axis_index("c")
    sid = lax.axis_index("s")
    wid = cc * 16 + sid
    frow0 = pl.multiple_of(sid * (RPS * 8), 8)
    pltpu.sync_copy(z8_hbm.at[pl.ds(frow0, RPS * 8)], z_v)
    pltpu.sync_copy(z_v, acc.at[pl.ds(frow0, RPS * 8)])
    plsc.subcore_barrier()
    for b in range(NB_A):
        base = pl.multiple_of(wid * EPW + b * BE_A, 8)
        fbase = pl.multiple_of(base * 8, 8)
        pltpu.sync_copy(sidxf_hbm.at[pl.ds(fbase, BE_A * 8)], sidx_v)
        pltpu.sync_copy(didxf_hbm.at[pl.ds(fbase, BE_A * 8)], didx_v)
        pltpu.async_copy(ssf_hbm.at[sidx_v], ra_v, sem).wait()
        pltpu.async_copy(sdf_hbm.at[didx_v], rb_v, sem).wait()

        def body(j, _):
            v = ra_v[pl.ds(j * 16, 16)] + rb_v[pl.ds(j * 16, 16)]
            v = jnp.maximum(v, 0.2 * v)
            p_v[pl.ds(j * 16, 16)] = jnp.exp(v)
            return 0

        lax.fori_loop(0, BE_A * 8 // 16, body, 0)
        pltpu.sync_copy(p_v, p_hbm.at[pl.ds(fbase, BE_A * 8)])
        pltpu.sync_copy(p_v, acc.at[didx_v], add=True)
    plsc.subcore_barrier()
    dof = pl.multiple_of(cc * (NP * 8) + frow0, 8)
    pltpu.sync_copy(acc.at[pl.ds(frow0, RPS * 8)], z_v)
    pltpu.sync_copy(z_v, den_hbm.at[pl.ds(dof, RPS * 8)])


def _scores(sidx_f, didx_f, ssf, sdf, z8):
    f32 = jnp.float32
    mesh = plsc.VectorSubcoreMesh(core_axis_name="c", subcore_axis_name="s")
    k = functools.partial(
        pl.kernel,
        mesh=mesh,
        out_type=[jax.ShapeDtypeStruct((EP * 8,), f32),
                  jax.ShapeDtypeStruct((2 * NP * 8,), f32)],
        scratch_types=[
            pltpu.VMEM((BE_A * 8,), jnp.int32),
            pltpu.VMEM((BE_A * 8,), jnp.int32),
            pltpu.VMEM((BE_A * 8,), f32),
            pltpu.VMEM((BE_A * 8,), f32),
            pltpu.VMEM((BE_A * 8,), f32),
            pltpu.VMEM((RPS * 8,), f32),
            pltpu.VMEM_SHARED((NP * 8,), f32),
            pltpu.SemaphoreType.DMA,
        ],
    )(_scores_body)
    return k(sidx_f, didx_f, ssf, sdf, z8)


def _attn_body(didxf_hbm, p_hbm, d0_hbm, d1_hbm,
               attn_hbm,
               didx_v, r0_v, r1_v, p_v, at_v, sem):
    cc = lax.axis_index("c")
    sid = lax.axis_index("s")
    wid = cc * 16 + sid
    for b in range(NB_A):
        base = pl.multiple_of(wid * EPW + b * BE_A, 8)
        fbase = pl.multiple_of(base * 8, 8)
        pltpu.sync_copy(didxf_hbm.at[pl.ds(fbase, BE_A * 8)], didx_v)
        pltpu.async_copy(d0_hbm.at[didx_v], r0_v, sem).wait()
        pltpu.async_copy(d1_hbm.at[didx_v], r1_v, sem).wait()
        pltpu.sync_copy(p_hbm.at[pl.ds(fbase, BE_A * 8)], p_v)

        def body(j, _):
            sl = pl.ds(j * 16, 16)
            at_v[sl] = p_v[sl] / (r0_v[sl] + r1_v[sl] + 1e-16)
            return 0

        lax.fori_loop(0, BE_A * 8 // 16, body, 0)
        pltpu.sync_copy(at_v, attn_hbm.at[pl.ds(fbase, BE_A * 8)])


def _attn(didx_f, p_hbm, den0, den1):
    f32 = jnp.float32
    mesh = plsc.VectorSubcoreMesh(core_axis_name="c", subcore_axis_name="s")
    k = functools.partial(
        pl.kernel,
        mesh=mesh,
        out_type=jax.ShapeDtypeStruct((EP * 8,), f32),
        scratch_types=[
            pltpu.VMEM((BE_A * 8,), jnp.int32),
            pltpu.VMEM((BE_A * 8,), f32),
            pltpu.VMEM((BE_A * 8,), f32),
            pltpu.VMEM((BE_A * 8,), f32),
            pltpu.VMEM((BE_A * 8,), f32),
            pltpu.SemaphoreType.DMA,
        ],
    )(_attn_body)
    return k(didx_f, p_hbm, den0, den1)


def _agg_body(src_hbm, dst_hbm, attn_hbm, *rest):
    projs = rest[:NCHUNK]
    z128_hbm = rest[NCHUNK]
    outs = rest[NCHUNK + 1:2 * NCHUNK + 1]
    src_v, dst_v, at_v, proj_v, acc, sem = rest[2 * NCHUNK + 1:]
    cc = lax.axis_index("c")
    sid = lax.axis_index("s")
    wid = cc * 16 + sid
    row0 = pl.multiple_of(sid * RPS, 8)
    for c in range(NCHUNK):
        pltpu.sync_copy(z128_hbm.at[pl.ds(row0, RPS)], acc.at[pl.ds(row0, RPS)])
        plsc.subcore_barrier()

        def batch(b, _, c=c):
            base = pl.multiple_of(wid * EPW + b * BE_B, 8)
            fbase = pl.multiple_of(base * 8, 8)
            pltpu.sync_copy(src_hbm.at[pl.ds(base, BE_B)], src_v)
            pltpu.sync_copy(dst_hbm.at[pl.ds(base, BE_B)], dst_v)
            pltpu.sync_copy(attn_hbm.at[pl.ds(fbase, BE_B * 8)],
                            at_v.at[pl.ds(0, BE_B * 8)])
            pltpu.async_copy(projs[c].at[src_v], proj_v, sem).wait()

            def body(j, _):
                for l in range(16):
                    e = j * 16 + l
                    for hh in range(2):
                        w = at_v[pl.ds(e * 8 + 2 * c + hh, 16)]
                        bvec = jnp.broadcast_to(w[0], (16,))
                        for kq in range(4):
                            off = hh * 64 + kq * 16
                            proj_v[e, pl.ds(off, 16)] = (
                                proj_v[e, pl.ds(off, 16)] * bvec)
                return 0

            lax.fori_loop(0, BE_B // 16, body, 0)
            pltpu.sync_copy(proj_v, acc.at[dst_v], add=True)
            return 0

        lax.fori_loop(0, NB_B2, batch, 0)
        plsc.subcore_barrier()
        oro = pl.multiple_of(cc * NP + row0, 8)
        pltpu.sync_copy(acc.at[pl.ds(row0, RPS)], outs[c].at[pl.ds(oro, RPS)])
        plsc.subcore_barrier()


def _aggregate(src_p, dst_p, attn_hbm, projs, z128):
    f32 = jnp.float32
    mesh = plsc.VectorSubcoreMesh(core_axis_name="c", subcore_axis_name="s")
    k = functools.partial(
        pl.kernel,
        mesh=mesh,
        out_type=[jax.ShapeDtypeStruct((2 * NP, 128), f32)] * NCHUNK,
        scratch_types=[
            pltpu.VMEM((BE_B,), jnp.int32),
            pltpu.VMEM((BE_B,), jnp.int32),
            pltpu.VMEM((BE_B * 8 + 16,), f32),
            pltpu.VMEM((BE_B, 128), f32),
            pltpu.VMEM_SHARED((NP, 128), f32),
            pltpu.SemaphoreType.DMA,
        ],
    )(_agg_body)
    return k(src_p, dst_p, attn_hbm, *projs, z128)


def _final_body(*refs):
    parts0 = refs[:NCHUNK]
    parts1 = refs[NCHUNK:2 * NCHUNK]
    skip_ref, o_ref = refs[2 * NCHUNK:]
    for c in range(NCHUNK):
        s = (parts0[c][...] + parts1[c][...]
             + skip_ref[:, c * 128:(c + 1) * 128])
        o_ref[:, c * 128:(c + 1) * 128] = jnp.where(
            s > 0, s, jnp.exp(jnp.minimum(s, 0.0)) - 1.0)


def _final(parts0, parts1, skip):
    return pl.pallas_call(
        _final_body,
        grid=(N // RB,),
        in_specs=[pl.BlockSpec((RB, 128), lambda i: (i, 0))] * (2 * NCHUNK)
        + [pl.BlockSpec((RB, 512), lambda i: (i, 0))],
        out_specs=pl.BlockSpec((RB, 512), lambda i: (i, 0)),
        out_shape=jax.ShapeDtypeStruct((N, 512), jnp.float32),
    )(*parts0, *parts1, skip)


def kernel(x, edge_index, W, a_src, a_dst, W_skip, bias):
    f32 = jnp.float32
    eye = jnp.eye(NH, dtype=f32)
    A_src = (a_src[:, :, None] * eye[:, None, :]).reshape(NH * 64, NH)
    A_dst = (a_dst[:, :, None] * eye[:, None, :]).reshape(NH * 64, NH)
    bias2d = bias.reshape(1, 512)

    dense_outs = _dense(x, W, W_skip, A_src, A_dst, bias2d)
    projs = tuple(dense_outs[:4])
    ssrc, sdst, skip = dense_outs[4:]

    pad = EP - E
    src_p = jnp.concatenate([edge_index[0], jnp.zeros((pad,), jnp.int32)])
    dst_p = jnp.concatenate([edge_index[1], jnp.full((pad,), N + 16, jnp.int32)])
    ar8 = jnp.arange(8, dtype=jnp.int32)
    sidx_f = (src_p[:, None] * 8 + ar8).reshape(-1)
    didx_f = (dst_p[:, None] * 8 + ar8).reshape(-1)
    z8 = jnp.zeros((NP * 8,), f32)
    z128 = jnp.zeros((NP, 128), f32)

    padn = (NP - N) * 8
    ssf = jnp.pad(ssrc.reshape(-1), (0, padn))
    sdf = jnp.pad(sdst.reshape(-1), (0, padn))
    p_hbm, den = _scores(sidx_f, didx_f, ssf, sdf, z8)
    attn_hbm = _attn(didx_f, p_hbm, den[:NP * 8], den[NP * 8:])
    parts = _aggregate(src_p, dst_p, attn_hbm, projs, z128)
    parts0 = [p[:N] for p in parts]
    parts1 = [p[NP:NP + N] for p in parts]
    return _final(parts0, parts1, skip)


# speedup vs baseline: 8.3262x; 1.0543x over previous
import functools

import jax
import jax.numpy as jnp
from jax import lax
from jax.experimental import pallas as pl
from jax.experimental.pallas import tpu as pltpu
from jax.experimental.pallas import tpu_sc as plsc

N = 10000
NP = 10112
E = 160000
EP = 163840
NH = 8
NW = 32
EPW = EP // NW
BE_A = 1024
NB_A = EPW // BE_A
BE_B = 256
NB_B2 = EPW // BE_B
NCHUNK = 4
RB = 400
RPS = NP // 16


def _dense_body(x_ref, w_ref, wsk_ref, asrc_ref, adst_ref, bias_ref,
                *out_refs):
    (p0_ref, p1_ref, p2_ref, p3_ref, ss_ref, sd_ref, skip_ref) = out_refs
    xb = x_ref[...]
    pr = jnp.dot(xb, w_ref[...], preferred_element_type=jnp.float32)
    for c, pref in enumerate((p0_ref, p1_ref, p2_ref, p3_ref)):
        pref[...] = pr[:, c * 128:(c + 1) * 128]
    ss_ref[...] = jnp.dot(pr, asrc_ref[...], preferred_element_type=jnp.float32)
    sd_ref[...] = jnp.dot(pr, adst_ref[...], preferred_element_type=jnp.float32)
    skip_ref[...] = (jnp.dot(xb, wsk_ref[...], preferred_element_type=jnp.float32)
                     + bias_ref[...])


def _dense(x, W, W_skip, A_src, A_dst, bias2d):
    f32 = jnp.float32
    return pl.pallas_call(
        _dense_body,
        grid=(N // RB,),
        in_specs=[
            pl.BlockSpec((RB, 256), lambda i: (i, 0)),
            pl.BlockSpec((256, 512), lambda i: (0, 0)),
            pl.BlockSpec((256, 512), lambda i: (0, 0)),
            pl.BlockSpec((512, 8), lambda i: (0, 0)),
            pl.BlockSpec((512, 8), lambda i: (0, 0)),
            pl.BlockSpec((1, 512), lambda i: (0, 0)),
        ],
        out_specs=[pl.BlockSpec((RB, 128), lambda i: (i, 0))] * 4 + [
            pl.BlockSpec((RB, 8), lambda i: (i, 0)),
            pl.BlockSpec((RB, 8), lambda i: (i, 0)),
            pl.BlockSpec((RB, 512), lambda i: (i, 0)),
        ],
        out_shape=[jax.ShapeDtypeStruct((N, 128), f32)] * 4 + [
            jax.ShapeDtypeStruct((N, 8), f32),
            jax.ShapeDtypeStruct((N, 8), f32),
            jax.ShapeDtypeStruct((N, 512), f32),
        ],
    )(x, W, W_skip, A_src, A_dst, bias2d)


def _scores_body(sidxf_hbm, didxf_hbm, ssf_hbm, sdf_hbm, z8_hbm,
                 p_hbm, den_hbm,
                 sidx_v, didx_v, ra_v, rb_v, p_v, z_v, acc, sem):
    cc = lax.axis_index("c")
    sid = lax.axis_index("s")
    wid = cc * 16 + sid
    frow0 = pl.multiple_of(sid * (RPS * 8), 8)
    pltpu.sync_copy(z8_hbm.at[pl.ds(frow0, RPS * 8)], z_v)
    pltpu.sync_copy(z_v, acc.at[pl.ds(frow0, RPS * 8)])
    plsc.subcore_barrier()
    for b in range(NB_A):
        base = pl.multiple_of(wid * EPW + b * BE_A, 8)
        fbase = pl.multiple_of(base * 8, 8)
        pltpu.sync_copy(sidxf_hbm.at[pl.ds(fbase, BE_A * 8)], sidx_v)
        pltpu.sync_copy(didxf_hbm.at[pl.ds(fbase, BE_A * 8)], didx_v)
        pltpu.async_copy(ssf_hbm.at[sidx_v], ra_v, sem).wait()
        pltpu.async_copy(sdf_hbm.at[didx_v], rb_v, sem).wait()

        def body(j, _):
            v = ra_v[pl.ds(j * 16, 16)] + rb_v[pl.ds(j * 16, 16)]
            v = jnp.maximum(v, 0.2 * v)
            p_v[pl.ds(j * 16, 16)] = jnp.exp(v)
            return 0

        lax.fori_loop(0, BE_A * 8 // 16, body, 0)
        pltpu.sync_copy(p_v, p_hbm.at[pl.ds(fbase, BE_A * 8)])
        pltpu.sync_copy(p_v, acc.at[didx_v], add=True)
    plsc.subcore_barrier()
    dof = pl.multiple_of(cc * (NP * 8) + frow0, 8)
    pltpu.sync_copy(acc.at[pl.ds(frow0, RPS * 8)], z_v)
    pltpu.sync_copy(z_v, den_hbm.at[pl.ds(dof, RPS * 8)])


def _scores(sidx_f, didx_f, ssf, sdf, z8):
    f32 = jnp.float32
    mesh = plsc.VectorSubcoreMesh(core_axis_name="c", subcore_axis_name="s")
    k = functools.partial(
        pl.kernel,
        mesh=mesh,
        out_type=[jax.ShapeDtypeStruct((EP * 8,), f32),
                  jax.ShapeDtypeStruct((2 * NP * 8,), f32)],
        scratch_types=[
            pltpu.VMEM((BE_A * 8,), jnp.int32),
            pltpu.VMEM((BE_A * 8,), jnp.int32),
            pltpu.VMEM((BE_A * 8,), f32),
            pltpu.VMEM((BE_A * 8,), f32),
            pltpu.VMEM((BE_A * 8,), f32),
            pltpu.VMEM((RPS * 8,), f32),
            pltpu.VMEM_SHARED((NP * 8,), f32),
            pltpu.SemaphoreType.DMA,
        ],
    )(_scores_body)
    return k(sidx_f, didx_f, ssf, sdf, z8)


def _attn_body(didxf_hbm, p_hbm, d0_hbm, d1_hbm,
               attn_hbm,
               didx_v, r0_v, r1_v, p_v, at_v, sem):
    cc = lax.axis_index("c")
    sid = lax.axis_index("s")
    wid = cc * 16 + sid
    for b in range(NB_A):
        base = pl.multiple_of(wid * EPW + b * BE_A, 8)
        fbase = pl.multiple_of(base * 8, 8)
        pltpu.sync_copy(didxf_hbm.at[pl.ds(fbase, BE_A * 8)], didx_v)
        pltpu.async_copy(d0_hbm.at[didx_v], r0_v, sem).wait()
        pltpu.async_copy(d1_hbm.at[didx_v], r1_v, sem).wait()
        pltpu.sync_copy(p_hbm.at[pl.ds(fbase, BE_A * 8)], p_v)

        def body(j, _):
            sl = pl.ds(j * 16, 16)
            at_v[sl] = p_v[sl] / (r0_v[sl] + r1_v[sl] + 1e-16)
            return 0

        lax.fori_loop(0, BE_A * 8 // 16, body, 0)
        pltpu.sync_copy(at_v, attn_hbm.at[pl.ds(fbase, BE_A * 8)])


def _attn(didx_f, p_hbm, den0, den1):
    f32 = jnp.float32
    mesh = plsc.VectorSubcoreMesh(core_axis_name="c", subcore_axis_name="s")
    k = functools.partial(
        pl.kernel,
        mesh=mesh,
        out_type=jax.ShapeDtypeStruct((EP * 8,), f32),
        scratch_types=[
            pltpu.VMEM((BE_A * 8,), jnp.int32),
            pltpu.VMEM((BE_A * 8,), f32),
            pltpu.VMEM((BE_A * 8,), f32),
            pltpu.VMEM((BE_A * 8,), f32),
            pltpu.VMEM((BE_A * 8,), f32),
            pltpu.SemaphoreType.DMA,
        ],
    )(_attn_body)
    return k(didx_f, p_hbm, den0, den1)


def _agg_body(src_hbm, dst_hbm, attn_hbm, *rest):
    projs = rest[:NCHUNK]
    z128_hbm = rest[NCHUNK]
    outs = rest[NCHUNK + 1:2 * NCHUNK + 1]
    src_v, dst_v, at_v, proj_v, acc, sem = rest[2 * NCHUNK + 1:]
    cc = lax.axis_index("c")
    sid = lax.axis_index("s")
    wid = cc * 16 + sid
    row0 = pl.multiple_of(sid * RPS, 8)
    for c in range(NCHUNK):
        pltpu.sync_copy(z128_hbm.at[pl.ds(row0, RPS)], acc.at[pl.ds(row0, RPS)])
        plsc.subcore_barrier()

        def batch(b, _, c=c):
            base = pl.multiple_of(wid * EPW + b * BE_B, 8)
            fbase = pl.multiple_of(base * 8, 8)
            pltpu.sync_copy(src_hbm.at[pl.ds(base, BE_B)], src_v)
            pltpu.sync_copy(dst_hbm.at[pl.ds(base, BE_B)], dst_v)
            pltpu.sync_copy(attn_hbm.at[pl.ds(fbase, BE_B * 8)],
                            at_v.at[pl.ds(0, BE_B * 8)])
            pltpu.async_copy(projs[c].at[src_v], proj_v, sem).wait()

            def body(j, _):
                for l in range(16):
                    e = j * 16 + l
                    for hh in range(2):
                        w = at_v[pl.ds(e * 8 + 2 * c + hh, 16)]
                        bvec = jnp.broadcast_to(w[0], (16,))
                        for kq in range(4):
                            off = hh * 64 + kq * 16
                            proj_v[e, pl.ds(off, 16)] = (
                                proj_v[e, pl.ds(off, 16)] * bvec)
                return 0

            lax.fori_loop(0, BE_B // 16, body, 0)
            pltpu.sync_copy(proj_v, acc.at[dst_v], add=True)
            return 0

        lax.fori_loop(0, NB_B2, batch, 0)
        plsc.subcore_barrier()
        oro = pl.multiple_of(cc * NP + row0, 8)
        pltpu.sync_copy(acc.at[pl.ds(row0, RPS)], outs[c].at[pl.ds(oro, RPS)])
        plsc.subcore_barrier()


def _aggregate(src_p, dst_p, attn_hbm, projs, z128):
    f32 = jnp.float32
    mesh = plsc.VectorSubcoreMesh(core_axis_name="c", subcore_axis_name="s")
    k = functools.partial(
        pl.kernel,
        mesh=mesh,
        out_type=[jax.ShapeDtypeStruct((2 * NP, 128), f32)] * NCHUNK,
        scratch_types=[
            pltpu.VMEM((BE_B,), jnp.int32),
            pltpu.VMEM((BE_B,), jnp.int32),
            pltpu.VMEM((BE_B * 8 + 16,), f32),
            pltpu.VMEM((BE_B, 128), f32),
            pltpu.VMEM_SHARED((NP, 128), f32),
            pltpu.SemaphoreType.DMA,
        ],
    )(_agg_body)
    return k(src_p, dst_p, attn_hbm, *projs, z128)


def _final_body(*refs):
    parts0 = refs[:NCHUNK]
    parts1 = refs[NCHUNK:2 * NCHUNK]
    skip_ref, o_ref = refs[2 * NCHUNK:]
    for c in range(NCHUNK):
        s = (parts0[c][...] + parts1[c][...]
             + skip_ref[:, c * 128:(c + 1) * 128])
        o_ref[:, c * 128:(c + 1) * 128] = jnp.where(
            s > 0, s, jnp.exp(jnp.minimum(s, 0.0)) - 1.0)


def _final(parts0, parts1, skip):
    return pl.pallas_call(
        _final_body,
        grid=(N // RB,),
        in_specs=[pl.BlockSpec((RB, 128), lambda i: (i, 0))] * (2 * NCHUNK)
        + [pl.BlockSpec((RB, 512), lambda i: (i, 0))],
        out_specs=pl.BlockSpec((RB, 512), lambda i: (i, 0)),
        out_shape=jax.ShapeDtypeStruct((N, 512), jnp.float32),
    )(*parts0, *parts1, skip)


def kernel(x, edge_index, W, a_src, a_dst, W_skip, bias):
    f32 = jnp.float32
    eye = jnp.eye(NH, dtype=f32)
    A_src = (a_src[:, :, None] * eye[:, None, :]).reshape(NH * 64, NH)
    A_dst = (a_dst[:, :, None] * eye[:, None, :]).reshape(NH * 64, NH)
    bias2d = bias.reshape(1, 512)

    dense_outs = _dense(x, W, W_skip, A_src, A_dst, bias2d)
    projs = tuple(dense_outs[:4])
    ssrc, sdst, skip = dense_outs[4:]

    pad = EP - E
    src_p = jnp.concatenate([edge_index[0], jnp.zeros((pad,), jnp.int32)])
    dst_p = jnp.concatenate([edge_index[1], jnp.full((pad,), N + 16, jnp.int32)])
    perm = jnp.arange(EP, dtype=jnp.int32).reshape(EPW, NW).T.reshape(-1)
    src_p = src_p[perm]
    dst_p = dst_p[perm]
    ar8 = jnp.arange(8, dtype=jnp.int32)
    sidx_f = (src_p[:, None] * 8 + ar8).reshape(-1)
    didx_f = (dst_p[:, None] * 8 + ar8).reshape(-1)
    z8 = jnp.zeros((NP * 8,), f32)
    z128 = jnp.zeros((NP, 128), f32)

    padn = (NP - N) * 8
    ssf = jnp.pad(ssrc.reshape(-1), (0, padn))
    sdf = jnp.pad(sdst.reshape(-1), (0, padn))
    p_hbm, den = _scores(sidx_f, didx_f, ssf, sdf, z8)
    attn_hbm = _attn(didx_f, p_hbm, den[:NP * 8], den[NP * 8:])
    parts = _aggregate(src_p, dst_p, attn_hbm, projs, z128)
    parts0 = [p[:N] for p in parts]
    parts1 = [p[NP:NP + N] for p in parts]
    return _final(parts0, parts1, skip)


# speedup vs baseline: 8.3368x; 1.0013x over previous
import functools

import jax
import jax.numpy as jnp
from jax import lax
from jax.experimental import pallas as pl
from jax.experimental.pallas import tpu as pltpu
from jax.experimental.pallas import tpu_sc as plsc

N = 10000
NP = 10112
E = 160000
EP = 163840
NH = 8
NW = 32
EPW = EP // NW
BE_A = 1024
NB_A = EPW // BE_A
BE_B = 256
NB_B2 = EPW // BE_B
NCHUNK = 4
RB = 400
RPS = NP // 16


def _dense_body(x_ref, w_ref, asrc_ref, adst_ref, *out_refs):
    (p0_ref, p1_ref, p2_ref, p3_ref, ss_ref, sd_ref) = out_refs
    xb = x_ref[...]
    pr = jnp.dot(xb, w_ref[...], preferred_element_type=jnp.float32)
    for c, pref in enumerate((p0_ref, p1_ref, p2_ref, p3_ref)):
        pref[...] = pr[:, c * 128:(c + 1) * 128]
    ss_ref[...] = jnp.dot(pr, asrc_ref[...], preferred_element_type=jnp.float32)
    sd_ref[...] = jnp.dot(pr, adst_ref[...], preferred_element_type=jnp.float32)


def _dense(x, W, A_src, A_dst):
    f32 = jnp.float32
    return pl.pallas_call(
        _dense_body,
        grid=(N // RB,),
        in_specs=[
            pl.BlockSpec((RB, 256), lambda i: (i, 0)),
            pl.BlockSpec((256, 512), lambda i: (0, 0)),
            pl.BlockSpec((512, 8), lambda i: (0, 0)),
            pl.BlockSpec((512, 8), lambda i: (0, 0)),
        ],
        out_specs=[pl.BlockSpec((RB, 128), lambda i: (i, 0))] * 4 + [
            pl.BlockSpec((RB, 8), lambda i: (i, 0)),
            pl.BlockSpec((RB, 8), lambda i: (i, 0)),
        ],
        out_shape=[jax.ShapeDtypeStruct((N, 128), f32)] * 4 + [
            jax.ShapeDtypeStruct((N, 8), f32),
            jax.ShapeDtypeStruct((N, 8), f32),
        ],
    )(x, W, A_src, A_dst)


def _skip_body(x_ref, wsk_ref, bias_ref, skip_ref):
    skip_ref[...] = (jnp.dot(x_ref[...], wsk_ref[...],
                             preferred_element_type=jnp.float32)
                     + bias_ref[...])


def _skip(x, W_skip, bias2d):
    return pl.pallas_call(
        _skip_body,
        grid=(N // RB,),
        in_specs=[
            pl.BlockSpec((RB, 256), lambda i: (i, 0)),
            pl.BlockSpec((256, 512), lambda i: (0, 0)),
            pl.BlockSpec((1, 512), lambda i: (0, 0)),
        ],
        out_specs=pl.BlockSpec((RB, 512), lambda i: (i, 0)),
        out_shape=jax.ShapeDtypeStruct((N, 512), jnp.float32),
    )(x, W_skip, bias2d)


def _scores_body(sidxf_hbm, didxf_hbm, ssf_hbm, sdf_hbm, z8_hbm,
                 p_hbm, den_hbm,
                 sidx_v, didx_v, ra_v, rb_v, p_v, z_v, acc, sem):
    cc = lax.axis_index("c")
    sid = lax.axis_index("s")
    wid = cc * 16 + sid
    frow0 = pl.multiple_of(sid * (RPS * 8), 8)
    pltpu.sync_copy(z8_hbm.at[pl.ds(frow0, RPS * 8)], z_v)
    pltpu.sync_copy(z_v, acc.at[pl.ds(frow0, RPS * 8)])
    plsc.subcore_barrier()
    for b in range(NB_A):
        base = pl.multiple_of(wid * EPW + b * BE_A, 8)
        fbase = pl.multiple_of(base * 8, 8)
        pltpu.sync_copy(sidxf_hbm.at[pl.ds(fbase, BE_A * 8)], sidx_v)
        pltpu.sync_copy(didxf_hbm.at[pl.ds(fbase, BE_A * 8)], didx_v)
        pltpu.async_copy(ssf_hbm.at[sidx_v], ra_v, sem).wait()
        pltpu.async_copy(sdf_hbm.at[didx_v], rb_v, sem).wait()

        def body(j, _):
            v = ra_v[pl.ds(j * 16, 16)] + rb_v[pl.ds(j * 16, 16)]
            v = jnp.maximum(v, 0.2 * v)
            p_v[pl.ds(j * 16, 16)] = jnp.exp(v)
            return 0

        lax.fori_loop(0, BE_A * 8 // 16, body, 0)
        pltpu.sync_copy(p_v, p_hbm.at[pl.ds(fbase, BE_A * 8)])
        pltpu.sync_copy(p_v, acc.at[didx_v], add=True)
    plsc.subcore_barrier()
    dof = pl.multiple_of(cc * (NP * 8) + frow0, 8)
    pltpu.sync_copy(acc.at[pl.ds(frow0, RPS * 8)], z_v)
    pltpu.sync_copy(z_v, den_hbm.at[pl.ds(dof, RPS * 8)])


def _scores(sidx_f, didx_f, ssf, sdf, z8):
    f32 = jnp.float32
    mesh = plsc.VectorSubcoreMesh(core_axis_name="c", subcore_axis_name="s")
    k = functools.partial(
        pl.kernel,
        mesh=mesh,
        out_type=[jax.ShapeDtypeStruct((EP * 8,), f32),
                  jax.ShapeDtypeStruct((2 * NP * 8,), f32)],
        scratch_types=[
            pltpu.VMEM((BE_A * 8,), jnp.int32),
            pltpu.VMEM((BE_A * 8,), jnp.int32),
            pltpu.VMEM((BE_A * 8,), f32),
            pltpu.VMEM((BE_A * 8,), f32),
            pltpu.VMEM((BE_A * 8,), f32),
            pltpu.VMEM((RPS * 8,), f32),
            pltpu.VMEM_SHARED((NP * 8,), f32),
            pltpu.SemaphoreType.DMA,
        ],
    )(_scores_body)
    return k(sidx_f, didx_f, ssf, sdf, z8)


def _attn_body(didxf_hbm, p_hbm, d0_hbm, d1_hbm,
               attn_hbm,
               didx_v, r0_v, r1_v, p_v, at_v, sem):
    cc = lax.axis_index("c")
    sid = lax.axis_index("s")
    wid = cc * 16 + sid
    for b in range(NB_A):
        base = pl.multiple_of(wid * EPW + b * BE_A, 8)
        fbase = pl.multiple_of(base * 8, 8)
        pltpu.sync_copy(didxf_hbm.at[pl.ds(fbase, BE_A * 8)], didx_v)
        pltpu.async_copy(d0_hbm.at[didx_v], r0_v, sem).wait()
        pltpu.async_copy(d1_hbm.at[didx_v], r1_v, sem).wait()
        pltpu.sync_copy(p_hbm.at[pl.ds(fbase, BE_A * 8)], p_v)

        def body(j, _):
            sl = pl.ds(j * 16, 16)
            at_v[sl] = p_v[sl] / (r0_v[sl] + r1_v[sl] + 1e-16)
            return 0

        lax.fori_loop(0, BE_A * 8 // 16, body, 0)
        pltpu.sync_copy(at_v, attn_hbm.at[pl.ds(fbase, BE_A * 8)])


def _attn(didx_f, p_hbm, den0, den1):
    f32 = jnp.float32
    mesh = plsc.VectorSubcoreMesh(core_axis_name="c", subcore_axis_name="s")
    k = functools.partial(
        pl.kernel,
        mesh=mesh,
        out_type=jax.ShapeDtypeStruct((EP * 8,), f32),
        scratch_types=[
            pltpu.VMEM((BE_A * 8,), jnp.int32),
            pltpu.VMEM((BE_A * 8,), f32),
            pltpu.VMEM((BE_A * 8,), f32),
            pltpu.VMEM((BE_A * 8,), f32),
            pltpu.VMEM((BE_A * 8,), f32),
            pltpu.SemaphoreType.DMA,
        ],
    )(_attn_body)
    return k(didx_f, p_hbm, den0, den1)


def _agg_body(src_hbm, dst_hbm, attn_hbm, *rest):
    projs = rest[:NCHUNK]
    z128_hbm = rest[NCHUNK]
    outs = rest[NCHUNK + 1:2 * NCHUNK + 1]
    src_v, dst_v, at_v, proj_v, acc, sem = rest[2 * NCHUNK + 1:]
    cc = lax.axis_index("c")
    sid = lax.axis_index("s")
    wid = cc * 16 + sid
    row0 = pl.multiple_of(sid * RPS, 8)
    for c in range(NCHUNK):
        pltpu.sync_copy(z128_hbm.at[pl.ds(row0, RPS)], acc.at[pl.ds(row0, RPS)])
        plsc.subcore_barrier()

        def batch(b, _, c=c):
            base = pl.multiple_of(wid * EPW + b * BE_B, 8)
            fbase = pl.multiple_of(base * 8, 8)
            pltpu.sync_copy(src_hbm.at[pl.ds(base, BE_B)], src_v)
            pltpu.sync_copy(dst_hbm.at[pl.ds(base, BE_B)], dst_v)
            pltpu.sync_copy(attn_hbm.at[pl.ds(fbase, BE_B * 8)],
                            at_v.at[pl.ds(0, BE_B * 8)])
            pltpu.async_copy(projs[c].at[src_v], proj_v, sem).wait()

            def body(j, _):
                for l in range(16):
                    e = j * 16 + l
                    for hh in range(2):
                        w = at_v[pl.ds(e * 8 + 2 * c + hh, 16)]
                        bvec = jnp.broadcast_to(w[0], (16,))
                        for kq in range(4):
                            off = hh * 64 + kq * 16
                            proj_v[e, pl.ds(off, 16)] = (
                                proj_v[e, pl.ds(off, 16)] * bvec)
                return 0

            lax.fori_loop(0, BE_B // 16, body, 0)
            pltpu.sync_copy(proj_v, acc.at[dst_v], add=True)
            return 0

        lax.fori_loop(0, NB_B2, batch, 0)
        plsc.subcore_barrier()
        oro = pl.multiple_of(cc * NP + row0, 8)
        pltpu.sync_copy(acc.at[pl.ds(row0, RPS)], outs[c].at[pl.ds(oro, RPS)])
        plsc.subcore_barrier()


def _aggregate(src_p, dst_p, attn_hbm, projs, z128):
    f32 = jnp.float32
    mesh = plsc.VectorSubcoreMesh(core_axis_name="c", subcore_axis_name="s")
    k = functools.partial(
        pl.kernel,
        mesh=mesh,
        out_type=[jax.ShapeDtypeStruct((2 * NP, 128), f32)] * NCHUNK,
        scratch_types=[
            pltpu.VMEM((BE_B,), jnp.int32),
            pltpu.VMEM((BE_B,), jnp.int32),
            pltpu.VMEM((BE_B * 8 + 16,), f32),
            pltpu.VMEM((BE_B, 128), f32),
            pltpu.VMEM_SHARED((NP, 128), f32),
            pltpu.SemaphoreType.DMA,
        ],
    )(_agg_body)
    return k(src_p, dst_p, attn_hbm, *projs, z128)


def _final_body(*refs):
    parts0 = refs[:NCHUNK]
    parts1 = refs[NCHUNK:2 * NCHUNK]
    skip_ref, o_ref = refs[2 * NCHUNK:]
    for c in range(NCHUNK):
        s = (parts0[c][...] + parts1[c][...]
             + skip_ref[:, c * 128:(c + 1) * 128])
        o_ref[:, c * 128:(c + 1) * 128] = jnp.where(
            s > 0, s, jnp.exp(jnp.minimum(s, 0.0)) - 1.0)


def _final(parts0, parts1, skip):
    return pl.pallas_call(
        _final_body,
        grid=(N // RB,),
        in_specs=[pl.BlockSpec((RB, 128), lambda i: (i, 0))] * (2 * NCHUNK)
        + [pl.BlockSpec((RB, 512), lambda i: (i, 0))],
        out_specs=pl.BlockSpec((RB, 512), lambda i: (i, 0)),
        out_shape=jax.ShapeDtypeStruct((N, 512), jnp.float32),
    )(*parts0, *parts1, skip)


def kernel(x, edge_index, W, a_src, a_dst, W_skip, bias):
    f32 = jnp.float32
    eye = jnp.eye(NH, dtype=f32)
    A_src = (a_src[:, :, None] * eye[:, None, :]).reshape(NH * 64, NH)
    A_dst = (a_dst[:, :, None] * eye[:, None, :]).reshape(NH * 64, NH)
    bias2d = bias.reshape(1, 512)

    dense_outs = _dense(x, W, A_src, A_dst)
    projs = tuple(dense_outs[:4])
    ssrc, sdst = dense_outs[4:]
    skip = _skip(x, W_skip, bias2d)

    pad = EP - E
    src_p = jnp.concatenate([edge_index[0], jnp.zeros((pad,), jnp.int32)])
    dst_p = jnp.concatenate([edge_index[1], jnp.full((pad,), N + 16, jnp.int32)])
    perm = jnp.arange(EP, dtype=jnp.int32).reshape(EPW, NW).T.reshape(-1)
    src_p = src_p[perm]
    dst_p = dst_p[perm]
    ar8 = jnp.arange(8, dtype=jnp.int32)
    sidx_f = (src_p[:, None] * 8 + ar8).reshape(-1)
    didx_f = (dst_p[:, None] * 8 + ar8).reshape(-1)
    z8 = jnp.zeros((NP * 8,), f32)
    z128 = jnp.zeros((NP, 128), f32)

    padn = (NP - N) * 8
    ssf = jnp.pad(ssrc.reshape(-1), (0, padn))
    sdf = jnp.pad(sdst.reshape(-1), (0, padn))
    p_hbm, den = _scores(sidx_f, didx_f, ssf, sdf, z8)
    attn_hbm = _attn(didx_f, p_hbm, den[:NP * 8], den[NP * 8:])
    parts = _aggregate(src_p, dst_p, attn_hbm, projs, z128)
    parts0 = [p[:N] for p in parts]
    parts1 = [p[NP:NP + N] for p in parts]
    return _final(parts0, parts1, skip)
